# baseline (speedup 1.0000x reference)
import functools

import jax
import jax.numpy as jnp
from jax import lax
from jax.experimental import pallas as pl
from jax.experimental.pallas import tpu as pltpu
from jax.experimental.pallas import tpu_sc as plsc

N_CODES_HALF = 1024
D = 64
K = 2 * N_CODES_HALF
T = 1024
GRID = 64
N_TOK = GRID * T
COMMITMENT_COST = 0.25

SC_CORES = 2
SC_SUBCORES = 16
NW = SC_CORES * SC_SUBCORES
B_PER_W = N_TOK // NW
CHUNK = 1024


def _tc_body(x_ref, codes2_ref, bias_ref, revk_ref, idx_ref, hist_ref,
             acc_ref):
    b = pl.program_id(0)

    @pl.when(b == 0)
    def _init():
        acc_ref[0, 0] = 0.0

    x = x_ref[0]
    xc2 = jnp.dot(codes2_ref[...], x, preferred_element_type=jnp.float32)
    d = bias_ref[...] - xc2
    dmin = jnp.min(d, axis=0, keepdims=True)
    eqf = (d == dmin).astype(jnp.float32)
    idxf = float(K) - jnp.max(revk_ref[...] * eqf, axis=0,
                              keepdims=True)
    idx_ref[0, 0, :] = idxf[0].astype(jnp.int32)
    hist_ref[0, 0, :] = lax.dot_general(
        jnp.ones((1, T), jnp.float32), eqf,
        (((1,), (1,)), ((), ())), preferred_element_type=jnp.float32)[0]
    xnorm = jnp.sum(x * x, axis=0, keepdims=True)
    acc_ref[0, 0] += jnp.sum(dmin + xnorm)


def _fin_body(h0_ref, h1_ref, a0_ref, a1_ref, loss_ref, perp_ref):
    loss_ref[0, 0] = ((1.0 + COMMITMENT_COST) / (N_TOK * D)) * (
        a0_ref[0, 0] + a1_ref[0, 0])
    p = (jnp.sum(h0_ref[:, 0, :], axis=0)
         + jnp.sum(h1_ref[:, 0, :], axis=0)) * (1.0 / N_TOK)
    perp_ref[0, 0] = jnp.exp(-jnp.sum(p * jnp.log(p + 1e-10)))


@functools.lru_cache(maxsize=2)
def _sc_gather_fn(n_tok):
    mesh = plsc.VectorSubcoreMesh(core_axis_name="c", subcore_axis_name="s")
    b_per_w = n_tok // NW
    chunk = min(b_per_w, CHUNK)

    @functools.partial(
        pl.kernel,
        out_type=jax.ShapeDtypeStruct((n_tok, D), jnp.float32),
        mesh=mesh,
        scratch_types=[
            pltpu.VMEM((chunk,), jnp.int32),
            pltpu.VMEM((chunk, D), jnp.float32),
            pltpu.SemaphoreType.DMA,
        ],
        compiler_params=pltpu.CompilerParams(use_tc_tiling_on_sc=False),
    )
    def _sc_gather(codes_hbm, idx_hbm, out_hbm, idx_v, rows_v, sem):
        wid = lax.axis_index("s") * SC_CORES + lax.axis_index("c")
        base = wid * b_per_w
        for j in range(b_per_w // chunk):
            off = base + j * chunk
            pltpu.sync_copy(idx_hbm.at[pl.ds(off, chunk)], idx_v)
            pltpu.async_copy(codes_hbm.at[idx_v], rows_v, sem).wait()
            pltpu.sync_copy(rows_v, out_hbm.at[pl.ds(off, chunk)])

    return _sc_gather


def kernel(inputs, embed0, embed1, idx):
    codes = jnp.concatenate([embed0, embed1], axis=0)
    cnorm = jnp.sum(codes * codes, axis=1, keepdims=True)
    half = (jnp.arange(K, dtype=jnp.int32) >= N_CODES_HALF)[:, None]
    pen = jnp.where(half & (idx == 0), jnp.inf, 0.0).astype(jnp.float32)
    bias = cnorm + pen
    revk = (float(K) - jnp.arange(K, dtype=jnp.float32))[:, None]

    x_all = inputs.reshape(GRID, D, T)
    hg = GRID // 2
    idx_h, hist_h, acc_h, q_h = [], [], [], []
    for h in range(2):
        indices, hist, acc = pl.pallas_call(
            _tc_body,
            grid=(hg,),
            in_specs=[
                pl.BlockSpec((1, D, T), lambda b, h=h: (b + h * hg, 0, 0)),
                pl.BlockSpec((K, D), lambda b: (0, 0)),
                pl.BlockSpec((K, 1), lambda b: (0, 0)),
                pl.BlockSpec((K, 1), lambda b: (0, 0)),
            ],
            out_specs=[
                pl.BlockSpec((1, 1, T), lambda b: (b, 0, 0)),
                pl.BlockSpec((1, 1, K), lambda b: (b, 0, 0)),
                pl.BlockSpec(block_shape=(1, 1), index_map=lambda b: (0, 0),
                             memory_space=pltpu.SMEM),
            ],
            out_shape=[
                jax.ShapeDtypeStruct((hg, 1, T), jnp.int32),
                jax.ShapeDtypeStruct((hg, 1, K), jnp.float32),
                jax.ShapeDtypeStruct((1, 1), jnp.float32),
            ],
        )(x_all, 2.0 * codes, bias, revk)
        idx_h.append(indices)
        hist_h.append(hist)
        acc_h.append(acc)
        quant = _sc_gather_fn(hg * T)(codes, indices.reshape(hg * T))
        q_h.append(quant.reshape(hg, 32, 32, D).transpose(0, 3, 1, 2))

    loss, perp = pl.pallas_call(
        _fin_body,
        in_specs=[
            pl.BlockSpec((hg, 1, K), lambda: (0, 0, 0)),
            pl.BlockSpec((hg, 1, K), lambda: (0, 0, 0)),
            pl.BlockSpec(block_shape=(1, 1), index_map=lambda: (0, 0),
                         memory_space=pltpu.SMEM),
            pl.BlockSpec(block_shape=(1, 1), index_map=lambda: (0, 0),
                         memory_space=pltpu.SMEM),
        ],
        out_specs=[
            pl.BlockSpec(block_shape=(1, 1), index_map=lambda: (0, 0),
                         memory_space=pltpu.SMEM),
            pl.BlockSpec(block_shape=(1, 1), index_map=lambda: (0, 0),
                         memory_space=pltpu.SMEM),
        ],
        out_shape=[
            jax.ShapeDtypeStruct((1, 1), jnp.float32),
            jax.ShapeDtypeStruct((1, 1), jnp.float32),
        ],
    )(hist_h[0], hist_h[1], acc_h[0], acc_h[1])

    q = jnp.zeros((GRID, D, 32, 32), jnp.float32)
    q = lax.dynamic_update_slice(q, q_h[0], (0, 0, 0, 0))
    q = lax.dynamic_update_slice(q, q_h[1], (hg, 0, 0, 0))
    return (q, loss[0, 0], perp[0, 0])


# speedup vs baseline: 1.0469x; 1.0469x over previous
import functools

import jax
import jax.numpy as jnp
from jax import lax
from jax.experimental import pallas as pl
from jax.experimental.pallas import tpu as pltpu
from jax.experimental.pallas import tpu_sc as plsc

N_CODES_HALF = 1024
D = 64
K = 2 * N_CODES_HALF
T = 1024
GRID = 64
N_TOK = GRID * T
COMMITMENT_COST = 0.25

SC_CORES = 2
SC_SUBCORES = 16
NW = SC_CORES * SC_SUBCORES
B_PER_W = N_TOK // NW
CHUNK = 1024


def _tc_body(x_ref, codes2_ref, bias_ref, revk_ref, idx_ref, hist_ref,
             acc_ref):
    b = pl.program_id(0)

    @pl.when(b == 0)
    def _init():
        acc_ref[0, 0] = 0.0

    x = x_ref[0]
    xc2 = jnp.dot(codes2_ref[...], x, preferred_element_type=jnp.float32)
    d = bias_ref[...] - xc2
    dmin = jnp.min(d, axis=0, keepdims=True)
    eqf = (d == dmin).astype(jnp.float32)
    idxf = float(K) - jnp.max(revk_ref[...] * eqf, axis=0,
                              keepdims=True)
    idx_ref[0, 0, :] = idxf[0].astype(jnp.int32)
    hist_ref[0, 0, :] = lax.dot_general(
        jnp.ones((1, T), jnp.float32), eqf,
        (((1,), (1,)), ((), ())), preferred_element_type=jnp.float32)[0]
    xnorm = jnp.sum(x * x, axis=0, keepdims=True)
    acc_ref[0, 0] += jnp.sum(dmin + xnorm)


def _fin_body(hist_ref, acc_ref, loss_ref, perp_ref):
    loss_ref[0, 0] = ((1.0 + COMMITMENT_COST) / (N_TOK * D)) * acc_ref[0, 0]
    p = jnp.sum(hist_ref[:, 0, :], axis=0) * (1.0 / N_TOK)
    perp_ref[0, 0] = jnp.exp(-jnp.sum(p * jnp.log(p + 1e-10)))


@functools.lru_cache(maxsize=2)
def _sc_gather_fn(n_tok):
    mesh = plsc.VectorSubcoreMesh(core_axis_name="c", subcore_axis_name="s")
    b_per_w = n_tok // NW
    chunk = min(b_per_w, CHUNK)

    @functools.partial(
        pl.kernel,
        out_type=jax.ShapeDtypeStruct((n_tok, D), jnp.float32),
        mesh=mesh,
        scratch_types=[
            pltpu.VMEM((chunk,), jnp.int32),
            pltpu.VMEM((chunk, D), jnp.float32),
            pltpu.SemaphoreType.DMA,
        ],
        compiler_params=pltpu.CompilerParams(use_tc_tiling_on_sc=False),
    )
    def _sc_gather(codes_hbm, idx_hbm, out_hbm, idx_v, rows_v, sem):
        wid = lax.axis_index("s") * SC_CORES + lax.axis_index("c")
        base = wid * b_per_w
        for j in range(b_per_w // chunk):
            off = base + j * chunk
            pltpu.sync_copy(idx_hbm.at[pl.ds(off, chunk)], idx_v)
            pltpu.async_copy(codes_hbm.at[idx_v], rows_v, sem).wait()
            pltpu.sync_copy(rows_v, out_hbm.at[pl.ds(off, chunk)])

    return _sc_gather


def kernel(inputs, embed0, embed1, idx):
    codes = jnp.concatenate([embed0, embed1], axis=0)
    cnorm = jnp.sum(codes * codes, axis=1, keepdims=True)
    half = (jnp.arange(K, dtype=jnp.int32) >= N_CODES_HALF)[:, None]
    pen = jnp.where(half & (idx == 0), jnp.inf, 0.0).astype(jnp.float32)
    bias = cnorm + pen
    revk = (float(K) - jnp.arange(K, dtype=jnp.float32))[:, None]

    indices, hist, acc = pl.pallas_call(
        _tc_body,
        grid=(GRID,),
        in_specs=[
            pl.BlockSpec((1, D, T), lambda b: (b, 0, 0)),
            pl.BlockSpec((K, D), lambda b: (0, 0)),
            pl.BlockSpec((K, 1), lambda b: (0, 0)),
            pl.BlockSpec((K, 1), lambda b: (0, 0)),
        ],
        out_specs=[
            pl.BlockSpec((1, 1, T), lambda b: (b, 0, 0)),
            pl.BlockSpec((1, 1, K), lambda b: (b, 0, 0)),
            pl.BlockSpec(block_shape=(1, 1), index_map=lambda b: (0, 0),
                         memory_space=pltpu.SMEM),
        ],
        out_shape=[
            jax.ShapeDtypeStruct((GRID, 1, T), jnp.int32),
            jax.ShapeDtypeStruct((GRID, 1, K), jnp.float32),
            jax.ShapeDtypeStruct((1, 1), jnp.float32),
        ],
    )(inputs.reshape(GRID, D, T), 2.0 * codes, bias, revk)

    loss, perp = pl.pallas_call(
        _fin_body,
        in_specs=[
            pl.BlockSpec((GRID, 1, K), lambda: (0, 0, 0)),
            pl.BlockSpec(block_shape=(1, 1), index_map=lambda: (0, 0),
                         memory_space=pltpu.SMEM),
        ],
        out_specs=[
            pl.BlockSpec(block_shape=(1, 1), index_map=lambda: (0, 0),
                         memory_space=pltpu.SMEM),
            pl.BlockSpec(block_shape=(1, 1), index_map=lambda: (0, 0),
                         memory_space=pltpu.SMEM),
        ],
        out_shape=[
            jax.ShapeDtypeStruct((1, 1), jnp.float32),
            jax.ShapeDtypeStruct((1, 1), jnp.float32),
        ],
    )(hist, acc)

    quantized = _sc_gather_fn(N_TOK)(codes, indices.reshape(N_TOK))
    q = quantized.reshape(GRID, 32, 32, D).transpose(0, 3, 1, 2)
    return (q, loss[0, 0], perp[0, 0])
